# baseline (speedup 1.0000x reference)
import jax
import jax.numpy as jnp
from jax.experimental import pallas as pl
from jax.experimental.pallas import tpu as pltpu
from jax.experimental.pallas import tpu_sc as plsc


_GATHER_WINDOW = 128


def _sc_gather_packed(emb4, ids4_row):
    n = ids4_row.shape[1]
    d = emb4.shape[1]

    @pl.kernel(
        out_type=jax.ShapeDtypeStruct((n, d), emb4.dtype),
        mesh=plsc.VectorSubcoreMesh(
            core_axis_name="core", subcore_axis_name="subcore"
        ),
    )
    def gather_kernel(emb_hbm, ids_hbm, out_hbm):
        def body(ids_vmem, out_vmem):
            pltpu.sync_copy(emb_hbm.at[ids_vmem.at[0]], out_vmem)

        pltpu.emit_pipeline(
            body,
            grid=(n // _GATHER_WINDOW,),
            in_specs=[
                pl.BlockSpec((1, _GATHER_WINDOW), index_map=lambda i: (0, i))
            ],
            out_specs=[
                pl.BlockSpec((_GATHER_WINDOW, d), index_map=lambda i: (i, 0))
            ],
            core_axis_name=("core", "subcore"),
            dimension_semantics=(pltpu.PARALLEL,),
        )(ids_hbm, out_hbm)

    return gather_kernel(emb4, ids4_row)


def _proj_kernel(x4_ref, off_ref, w_ref, b_ref, o_ref):
    x4 = x4_ref[...]
    off = off_ref[...]
    d = x4.shape[1] // 4
    x = jnp.zeros((x4.shape[0], d), jnp.float32)
    for k in range(4):
        x = x + jnp.where(off == k, 1.0, 0.0) * x4[:, k * d:(k + 1) * d]
    o_ref[...] = (
        jax.lax.dot_general(
            x,
            w_ref[...],
            (((1,), (1,)), ((), ())),
            preferred_element_type=jnp.float32,
        )
        + b_ref[...]
    )


def _tc_project(x4, off_col, W, b2d, block_v):
    n = x4.shape[0]
    v, d = W.shape
    grid = pl.cdiv(v, block_v)
    return pl.pallas_call(
        _proj_kernel,
        grid=(grid,),
        in_specs=[
            pl.BlockSpec((n, 128), lambda i: (0, 0)),
            pl.BlockSpec((n, 1), lambda i: (0, 0)),
            pl.BlockSpec((block_v, d), lambda i: (i, 0)),
            pl.BlockSpec((1, block_v), lambda i: (0, i)),
        ],
        out_specs=pl.BlockSpec((n, block_v), lambda i: (0, i)),
        out_shape=jax.ShapeDtypeStruct((n, v), jnp.float32),
    )(x4, off_col, W, b2d)


def kernel(input_ids, emb, W, b):
    bsz, seq = input_ids.shape
    n = bsz * seq
    ids = input_ids.reshape(n).astype(jnp.int32)
    emb4 = emb.reshape(emb.shape[0] // 4, 4 * emb.shape[1])
    x4 = _sc_gather_packed(emb4, (ids // 4).reshape(1, n))
    off_col = (ids % 4).reshape(n, 1)
    logits = _tc_project(x4, off_col, W, b.reshape(1, -1), block_v=2048)
    return logits.reshape(bsz, seq, W.shape[0])


# speedup vs baseline: 1.0934x; 1.0934x over previous
import jax
import jax.numpy as jnp
from jax import lax
from jax.experimental import pallas as pl
from jax.experimental.pallas import tpu as pltpu
from jax.experimental.pallas import tpu_sc as plsc


_NC, _NS = 2, 16


def _sc_gather(emb, ids):
    n = ids.shape[0]
    v, d = emb.shape
    half = n // _NC

    @pl.kernel(
        out_type=jax.ShapeDtypeStruct((n, d), emb.dtype),
        mesh=plsc.ScalarSubcoreMesh(axis_name="c", num_cores=_NC),
        scratch_types=[
            pltpu.SMEM((n,), jnp.int32),
            pltpu.SemaphoreType.DMA,
            pltpu.SemaphoreType.DMA,
        ],
    )
    def gather_kernel(table_hbm, idx_hbm, out_hbm, idx_s, isem, sem):
        core = lax.axis_index("c")
        base = core * half
        pltpu.async_copy(idx_hbm, idx_s, isem).wait()

        @pl.loop(0, half)
        def _issue(i):
            j = base + i
            pltpu.async_copy(table_hbm.at[idx_s[j]], out_hbm.at[j], sem)

        @pl.loop(0, half)
        def _drain(i):
            pltpu.make_async_copy(
                table_hbm.at[0], out_hbm.at[base], sem
            ).wait()

    return gather_kernel(emb, ids)


def _proj_kernel(x_ref, w_ref, b_ref, o_ref):
    o_ref[...] = (
        lax.dot_general(
            x_ref[...],
            w_ref[...],
            (((1,), (1,)), ((), ())),
            preferred_element_type=jnp.float32,
        )
        + b_ref[...]
    )


def _tc_project(x, W, b2d, block_v):
    n, d = x.shape
    v = W.shape[0]
    grid = pl.cdiv(v, block_v)
    return pl.pallas_call(
        _proj_kernel,
        grid=(grid,),
        in_specs=[
            pl.BlockSpec((n, d), lambda i: (0, 0)),
            pl.BlockSpec((block_v, d), lambda i: (i, 0)),
            pl.BlockSpec((1, block_v), lambda i: (0, i)),
        ],
        out_specs=pl.BlockSpec((n, block_v), lambda i: (0, i)),
        out_shape=jax.ShapeDtypeStruct((n, v), jnp.float32),
    )(x, W, b2d)


def kernel(input_ids, emb, W, b):
    bsz, seq = input_ids.shape
    n = bsz * seq
    ids = input_ids.reshape(n).astype(jnp.int32)
    x = _sc_gather(emb, ids)
    logits = _tc_project(x, W, b.reshape(1, -1), block_v=2048)
    return logits.reshape(bsz, seq, W.shape[0])
